# baseline (speedup 1.0000x reference)
import functools

import jax
import jax.numpy as jnp
from jax import lax
from jax.experimental import pallas as pl
from jax.experimental.pallas import tpu as pltpu

_THRESH = 0.35667494393873245
_N_MIN = 131072
_IGNORE = 255

_HBLK = 256


def _ce_body(x_ref, lbl_ref):
    x = x_ref[0]
    lbl = lbl_ref[0]
    s = jnp.sum(jnp.exp(jnp.minimum(x, 60.0)), axis=0)
    cidx = lax.broadcasted_iota(jnp.int32, x.shape, 0)
    x_lbl = jnp.sum(jnp.where(cidx == lbl[None], x, 0.0), axis=0)
    loss = jnp.log(s) - x_lbl
    return jnp.where(lbl == _IGNORE, 0.0, loss)


def _stats_kernel(x_ref, lbl_ref, sum_ref, cnt_ref):
    i = pl.program_id(0)

    @pl.when(i == 0)
    def _init():
        sum_ref[...] = jnp.zeros((1, 1), jnp.float32)
        cnt_ref[...] = jnp.zeros((1, 1), jnp.float32)

    loss = _ce_body(x_ref, lbl_ref)
    gt = loss > _THRESH
    sum_ref[...] += jnp.sum(jnp.where(gt, loss, 0.0))[None, None]
    cnt_ref[...] += jnp.sum(gt.astype(jnp.float32))[None, None]


def _loss_kernel(x_ref, lbl_ref, loss_ref):
    loss_ref[0] = _ce_body(x_ref, lbl_ref)


def _topk_kernel(loss_ref, out_ref):
    loss = jnp.maximum(loss_ref[...], 0.0)
    bits = lax.bitcast_convert_type(loss, jnp.int32)
    k0 = jnp.int32(_N_MIN)

    def body(r, carry):
        i = 31 - r
        prefix, k = carry
        pat = lax.shift_right_logical(prefix, i) | 1
        hit = lax.shift_right_logical(bits, i) == pat
        cnt1 = jnp.sum(hit.astype(jnp.int32))
        take = cnt1 >= k
        prefix = jnp.where(take, prefix | (1 << i), prefix)
        k = jnp.where(take, k, k - cnt1)
        return prefix, k

    prefix, _ = lax.fori_loop(0, 32, body, (jnp.int32(0), k0))
    t = lax.bitcast_convert_type(prefix, jnp.float32)
    gt = bits > prefix
    cnt_gt = jnp.sum(gt.astype(jnp.float32))
    sum_gt = jnp.sum(jnp.where(gt, loss, 0.0))
    kf = jnp.float32(_N_MIN)
    out_ref[...] = ((sum_gt + t * (kf - cnt_gt)) / kf)[None, None]


def kernel(logits, labels):
    n, c, h, w = logits.shape
    lbl = labels.astype(jnp.int32)
    nsteps = h // _HBLK
    grid = (n * nsteps,)

    def xmap(i):
        return (i // nsteps, 0, i % nsteps, 0)

    def lmap(i):
        return (i // nsteps, i % nsteps, 0)

    sum_gt, cnt_gt = pl.pallas_call(
        _stats_kernel,
        grid=grid,
        in_specs=[
            pl.BlockSpec((1, c, _HBLK, w), xmap),
            pl.BlockSpec((1, _HBLK, w), lmap),
        ],
        out_specs=[
            pl.BlockSpec((1, 1), lambda i: (0, 0)),
            pl.BlockSpec((1, 1), lambda i: (0, 0)),
        ],
        out_shape=[
            jax.ShapeDtypeStruct((1, 1), jnp.float32),
            jax.ShapeDtypeStruct((1, 1), jnp.float32),
        ],
    )(logits, lbl)

    s = sum_gt[0, 0]
    cnt = cnt_gt[0, 0]
    cond = cnt > _N_MIN + 0.5
    mean_thresh = s / jnp.maximum(cnt, 1.0)

    def fallback(_):
        loss = pl.pallas_call(
            _loss_kernel,
            grid=grid,
            in_specs=[
                pl.BlockSpec((1, c, _HBLK, w), xmap),
                pl.BlockSpec((1, _HBLK, w), lmap),
            ],
            out_specs=pl.BlockSpec((1, _HBLK, w), lmap),
            out_shape=jax.ShapeDtypeStruct((n, h, w), jnp.float32),
        )(logits, lbl)
        loss2 = loss.reshape(n * h, w)
        res = pl.pallas_call(
            _topk_kernel,
            out_shape=jax.ShapeDtypeStruct((1, 1), jnp.float32),
        )(loss2)
        return res[0, 0]

    return lax.cond(cond, lambda _: mean_thresh, fallback, None)
